# baseline (speedup 1.0000x reference)
import functools

import jax
import jax.numpy as jnp
import numpy as np
from jax.experimental import pallas as pl

N_RAYS = 65536
N_SAMPLES = 128
NEAR = 0.0
FAR = 100.0
BLOCK = 8192

_T = (np.arange(N_SAMPLES, dtype=np.float32) *
      np.float32(1.0 / (N_SAMPLES - 1)))
_OMT = np.float32(1.0) - _T
_M = np.zeros((16, 4 * N_SAMPLES), dtype=np.float32)
_M[0, 0:128] = _OMT
_M[1, 0:128] = _T
for _k in range(3):
    _lo = 128 * (_k + 1)
    _M[2 + _k, _lo:_lo + 128] = 1.0
    _M[5 + _k, _lo:_lo + 128] = _OMT
    _M[8 + _k, _lo:_lo + 128] = _T


def _ray_cast_kernel(o_ref, d_ref, c_ref, m_ref, pts_ref, z_ref, near_ref,
                     far_ref):
    o = o_ref[...]
    d = d_ref[...]
    c = c_ref[...]
    ox, oz = o[0:1], o[2:3]
    dx, dz = d[0:1], d[2:3]
    cx, cz, rad = c[0:1], c[1:2], c[2:3]

    norm = jnp.sqrt(dx * dx + dz * dz)
    ddnx = dx / norm
    ddnz = dz / norm
    ocx = cx - ox
    ocz = cz - oz
    oc_proj = ocx * ddnx + ocz * ddnz
    d2 = jnp.maximum(ocx * ocx + ocz * ocz - oc_proj * oc_proj, 0.0)
    half = jnp.sqrt(jnp.maximum(rad * rad - d2, 1e-8))
    new_near = (oc_proj - half) / norm
    new_far = (oc_proj + half) / norm
    invalid = jnp.sqrt(d2) > rad
    near = jnp.where(invalid, NEAR, new_near)
    far = jnp.where(invalid, FAR, new_far)
    near_ref[...] = near
    far_ref[...] = far

    at = jnp.concatenate(
        [near, far, o, d * near, d * far,
         jnp.zeros((5, near.shape[1]), jnp.float32)],
        axis=0,
    )
    r = jax.lax.dot_general(
        at,
        m_ref[...],
        (((0,), (0,)), ((), ())),
        preferred_element_type=jnp.float32,
        precision=jax.lax.Precision.DEFAULT,
    )
    z_ref[...] = r[:, 0:N_SAMPLES]
    pts_ref[0] = r[:, N_SAMPLES:2 * N_SAMPLES]
    pts_ref[1] = r[:, 2 * N_SAMPLES:3 * N_SAMPLES]
    pts_ref[2] = r[:, 3 * N_SAMPLES:4 * N_SAMPLES]


@functools.partial(jax.jit, static_argnames=())
def kernel(rays_o, rays_d, cyls, skts):
    del skts
    n = rays_o.shape[0]
    grid = (n // BLOCK,)
    in_spec = pl.BlockSpec((3, BLOCK), lambda i: (0, i))
    m_spec = pl.BlockSpec((16, 4 * N_SAMPLES), lambda i: (0, 0))
    row_spec = pl.BlockSpec((1, BLOCK), lambda i: (0, i))
    pts_t, z_vals, near_row, far_row = pl.pallas_call(
        _ray_cast_kernel,
        grid=grid,
        in_specs=[in_spec, in_spec, in_spec, m_spec],
        out_specs=[
            pl.BlockSpec((3, BLOCK, N_SAMPLES), lambda i: (0, i, 0)),
            pl.BlockSpec((BLOCK, N_SAMPLES), lambda i: (i, 0)),
            row_spec,
            row_spec,
        ],
        out_shape=[
            jax.ShapeDtypeStruct((3, n, N_SAMPLES), jnp.float32),
            jax.ShapeDtypeStruct((n, N_SAMPLES), jnp.float32),
            jax.ShapeDtypeStruct((1, n), jnp.float32),
            jax.ShapeDtypeStruct((1, n), jnp.float32),
        ],
    )(rays_o.T, rays_d.T, cyls.T, jnp.asarray(_M))
    pts = jnp.transpose(pts_t, (1, 2, 0))
    return (pts, z_vals, near_row.reshape(n, 1), far_row.reshape(n, 1))


# speedup vs baseline: 1.0324x; 1.0324x over previous
import functools

import jax
import jax.numpy as jnp
import numpy as np
from jax.experimental import pallas as pl

N_RAYS = 65536
N_SAMPLES = 128
NEAR = 0.0
FAR = 100.0
BLOCK = 4096

_T = (np.arange(N_SAMPLES, dtype=np.float32) *
      np.float32(1.0 / (N_SAMPLES - 1)))
_OMT = np.float32(1.0) - _T
_M = np.zeros((16, 4 * N_SAMPLES), dtype=np.float32)
_M[0, 0:128] = _OMT
_M[1, 0:128] = _T
for _k in range(3):
    _lo = 128 * (_k + 1)
    _M[2 + _k, _lo:_lo + 128] = 1.0
    _M[5 + _k, _lo:_lo + 128] = _OMT
    _M[8 + _k, _lo:_lo + 128] = _T


def _ray_cast_kernel(o_ref, d_ref, c_ref, m_ref, pts_ref, z_ref, near_ref,
                     far_ref):
    o = o_ref[...]
    d = d_ref[...]
    c = c_ref[...]
    ox, oz = o[0:1], o[2:3]
    dx, dz = d[0:1], d[2:3]
    cx, cz, rad = c[0:1], c[1:2], c[2:3]

    norm = jnp.sqrt(dx * dx + dz * dz)
    ddnx = dx / norm
    ddnz = dz / norm
    ocx = cx - ox
    ocz = cz - oz
    oc_proj = ocx * ddnx + ocz * ddnz
    d2 = jnp.maximum(ocx * ocx + ocz * ocz - oc_proj * oc_proj, 0.0)
    half = jnp.sqrt(jnp.maximum(rad * rad - d2, 1e-8))
    new_near = (oc_proj - half) / norm
    new_far = (oc_proj + half) / norm
    invalid = jnp.sqrt(d2) > rad
    near = jnp.where(invalid, NEAR, new_near)
    far = jnp.where(invalid, FAR, new_far)
    near_ref[...] = near
    far_ref[...] = far

    at = jnp.concatenate(
        [near, far, o, d * near, d * far,
         jnp.zeros((5, near.shape[1]), jnp.float32)],
        axis=0,
    )
    r = jax.lax.dot_general(
        at,
        m_ref[...],
        (((0,), (0,)), ((), ())),
        preferred_element_type=jnp.float32,
        precision=jax.lax.Precision.DEFAULT,
    )
    z_ref[...] = r[:, 0:N_SAMPLES]
    pts_ref[0] = r[:, N_SAMPLES:2 * N_SAMPLES]
    pts_ref[1] = r[:, 2 * N_SAMPLES:3 * N_SAMPLES]
    pts_ref[2] = r[:, 3 * N_SAMPLES:4 * N_SAMPLES]


@functools.partial(jax.jit, static_argnames=())
def kernel(rays_o, rays_d, cyls, skts):
    del skts
    n = rays_o.shape[0]
    grid = (n // BLOCK,)
    in_spec = pl.BlockSpec((3, BLOCK), lambda i: (0, i))
    m_spec = pl.BlockSpec((16, 4 * N_SAMPLES), lambda i: (0, 0))
    row_spec = pl.BlockSpec((1, BLOCK), lambda i: (0, i))
    pts_t, z_vals, near_row, far_row = pl.pallas_call(
        _ray_cast_kernel,
        grid=grid,
        in_specs=[in_spec, in_spec, in_spec, m_spec],
        out_specs=[
            pl.BlockSpec((3, BLOCK, N_SAMPLES), lambda i: (0, i, 0)),
            pl.BlockSpec((BLOCK, N_SAMPLES), lambda i: (i, 0)),
            row_spec,
            row_spec,
        ],
        out_shape=[
            jax.ShapeDtypeStruct((3, n, N_SAMPLES), jnp.float32),
            jax.ShapeDtypeStruct((n, N_SAMPLES), jnp.float32),
            jax.ShapeDtypeStruct((1, n), jnp.float32),
            jax.ShapeDtypeStruct((1, n), jnp.float32),
        ],
    )(rays_o.T, rays_d.T, cyls.T, jnp.asarray(_M))
    pts = jnp.transpose(pts_t, (1, 2, 0))
    return (pts, z_vals, near_row.reshape(n, 1), far_row.reshape(n, 1))
